# Initial kernel scaffold; baseline (speedup 1.0000x reference)
import jax
import jax.numpy as jnp
from jax.experimental import pallas as pl


def kernel(position, W):
    raise NotImplementedError("write your pallas kernel here")


# speedup vs baseline: 188.9810x; 188.9810x over previous
import jax
import jax.numpy as jnp
from jax import lax
from jax.experimental import pallas as pl
from jax.experimental.pallas import tpu as pltpu
from jax.experimental.pallas import tpu_sc as plsc

NC, NS, L = 2, 16, 16
NW = NC * NS
B, H, P = 16384, 200, 200
N = B * H
PER_W = N // NW


def _body(pos_hbm, w_hbm, out_hbm, table_v, buf_v):
    wid = lax.axis_index("s") * NC + lax.axis_index("c")
    base = wid * PER_W
    pltpu.sync_copy(w_hbm, table_v)
    pltpu.sync_copy(pos_hbm.at[pl.ds(base, PER_W)], buf_v)

    def it(i, carry):
        sl = pl.ds(i * L, L)
        buf_v[sl] = plsc.load_gather(table_v, [buf_v[sl]])
        return carry

    lax.fori_loop(0, PER_W // L, it, 0)
    pltpu.sync_copy(buf_v, out_hbm.at[pl.ds(base, PER_W)])


_mesh = plsc.VectorSubcoreMesh(
    core_axis_name="c", subcore_axis_name="s", num_cores=NC, num_subcores=NS
)

_lookup = pl.kernel(
    _body,
    out_type=jax.ShapeDtypeStruct((N,), jnp.int32),
    mesh=_mesh,
    compiler_params=pltpu.CompilerParams(needs_layout_passes=False),
    scratch_types=[
        pltpu.VMEM((P,), jnp.int32),
        pltpu.VMEM((PER_W,), jnp.int32),
    ],
)


def kernel(position, W):
    w_bits = lax.bitcast_convert_type(W.reshape(P), jnp.int32)
    out = _lookup(position.reshape(N), w_bits)
    return lax.bitcast_convert_type(out, jnp.float32).reshape(B, H)


# speedup vs baseline: 269.1497x; 1.4242x over previous
import jax
import jax.numpy as jnp
from jax import lax
from jax.experimental import pallas as pl
from jax.experimental.pallas import tpu as pltpu
from jax.experimental.pallas import tpu_sc as plsc

NC, NS, L = 2, 16, 16
NW = NC * NS
B, H, P = 16384, 200, 200
N = B * H
PER_W = N // NW
NBUF = 4
NCH = 8
CH = PER_W // NCH
UNROLL = 8


def _body(pos_hbm, w_hbm, out_hbm, table_v, idx_v, val_v, in_sems, out_sems):
    wid = lax.axis_index("s") * NC + lax.axis_index("c")
    base = wid * PER_W
    pltpu.sync_copy(w_hbm, table_v)

    ins = [
        pltpu.async_copy(
            pos_hbm.at[pl.ds(base + b * CH, CH)], idx_v.at[b], in_sems.at[b]
        )
        for b in range(NBUF)
    ]
    outs = [None] * NBUF
    for g in range(NCH):
        b = g % NBUF
        ins[b].wait()
        if outs[b] is not None:
            outs[b].wait()

        @plsc.parallel_loop(0, CH // L, unroll=UNROLL)
        def _gather(i, b=b):
            sl = pl.ds(i * L, L)
            val_v[b, sl] = plsc.load_gather(table_v, [idx_v[b, sl]])

        outs[b] = pltpu.async_copy(
            val_v.at[b], out_hbm.at[pl.ds(base + g * CH, CH)], out_sems.at[b]
        )
        if g + NBUF < NCH:
            ins[b] = pltpu.async_copy(
                pos_hbm.at[pl.ds(base + (g + NBUF) * CH, CH)],
                idx_v.at[b],
                in_sems.at[b],
            )
    for b in range(NBUF):
        outs[b].wait()


_mesh = plsc.VectorSubcoreMesh(
    core_axis_name="c", subcore_axis_name="s", num_cores=NC, num_subcores=NS
)

_lookup = pl.kernel(
    _body,
    out_type=jax.ShapeDtypeStruct((N,), jnp.int32),
    mesh=_mesh,
    compiler_params=pltpu.CompilerParams(needs_layout_passes=False),
    scratch_types=[
        pltpu.VMEM((P,), jnp.int32),
        pltpu.VMEM((NBUF, CH), jnp.int32),
        pltpu.VMEM((NBUF, CH), jnp.int32),
        pltpu.SemaphoreType.DMA((NBUF,)),
        pltpu.SemaphoreType.DMA((NBUF,)),
    ],
)


def kernel(position, W):
    w_bits = lax.bitcast_convert_type(W.reshape(P), jnp.int32)
    out = _lookup(position.reshape(N), w_bits)
    return lax.bitcast_convert_type(out, jnp.float32).reshape(B, H)


# speedup vs baseline: 884.9710x; 3.2880x over previous
import jax
import jax.numpy as jnp
from jax import lax
from jax.experimental import pallas as pl
from jax.experimental.pallas import tpu as pltpu
from jax.experimental.pallas import tpu_sc as plsc

NC, NS, L = 2, 16, 16
NW = NC * NS
B, H, P = 16384, 200, 200
N = B * H
PER_W = N // NW
NBUF = 4
NCH = 8
CH = PER_W // NCH
UNROLL = 8


def _body(pos_hbm, w_hbm, out_hbm, table_v, idx_v, val_v, in_sems, out_sems):
    wid = lax.axis_index("s") * NC + lax.axis_index("c")
    base = wid * PER_W
    pltpu.sync_copy(w_hbm, table_v)

    ins = [
        pltpu.async_copy(
            pos_hbm.at[pl.ds(base + b * CH, CH)], idx_v.at[b], in_sems.at[b]
        )
        for b in range(NBUF)
    ]
    outs = [None] * NBUF
    for g in range(NCH):
        b = g % NBUF
        ins[b].wait()
        if outs[b] is not None:
            outs[b].wait()

        @plsc.parallel_loop(0, CH // L, unroll=UNROLL)
        def _gather(i, b=b):
            sl = pl.ds(i * L, L)
            val_v[b, sl] = plsc.load_gather(table_v, [idx_v[b, sl]])

        outs[b] = pltpu.async_copy(
            val_v.at[b], out_hbm.at[pl.ds(base + g * CH, CH)], out_sems.at[b]
        )
        if g + NBUF < NCH:
            ins[b] = pltpu.async_copy(
                pos_hbm.at[pl.ds(base + (g + NBUF) * CH, CH)],
                idx_v.at[b],
                in_sems.at[b],
            )
    for b in range(NBUF):
        outs[b].wait()


_mesh = plsc.VectorSubcoreMesh(
    core_axis_name="c", subcore_axis_name="s", num_cores=NC, num_subcores=NS
)

_lookup = pl.kernel(
    _body,
    out_type=jax.ShapeDtypeStruct((N,), jnp.float32),
    mesh=_mesh,
    compiler_params=pltpu.CompilerParams(needs_layout_passes=False),
    scratch_types=[
        pltpu.VMEM((P,), jnp.float32),
        pltpu.VMEM((NBUF, CH), jnp.int32),
        pltpu.VMEM((NBUF, CH), jnp.float32),
        pltpu.SemaphoreType.DMA((NBUF,)),
        pltpu.SemaphoreType.DMA((NBUF,)),
    ],
)


def kernel(position, W):
    x = position.T.reshape(H // 8, 8, B // 128, 128).swapaxes(1, 2).reshape(N)
    y = _lookup(x, W.reshape(P))
    return y.reshape(H // 8, B // 128, 8, 128).swapaxes(1, 2).reshape(H, B).T


# speedup vs baseline: 890.9219x; 1.0067x over previous
import jax
import jax.numpy as jnp
from jax import lax
from jax.experimental import pallas as pl
from jax.experimental.pallas import tpu as pltpu
from jax.experimental.pallas import tpu_sc as plsc

NC, NS, L = 2, 16, 16
NW = NC * NS
B, H, P = 16384, 200, 200
N = B * H
PER_W = N // NW
NBUF = 8
NCH = 16
CH = PER_W // NCH
UNROLL = 8


def _body(pos_hbm, w_hbm, out_hbm, table_v, idx_v, val_v, in_sems, out_sems, w_sem):
    wid = lax.axis_index("s") * NC + lax.axis_index("c")
    base = wid * PER_W
    w_copy = pltpu.async_copy(w_hbm, table_v, w_sem)

    ins = [
        pltpu.async_copy(
            pos_hbm.at[pl.ds(base + b * CH, CH)], idx_v.at[b], in_sems.at[b]
        )
        for b in range(NBUF)
    ]
    outs = [None] * NBUF
    w_copy.wait()
    for g in range(NCH):
        b = g % NBUF
        ins[b].wait()
        if outs[b] is not None:
            outs[b].wait()

        @plsc.parallel_loop(0, CH // L, unroll=UNROLL)
        def _gather(i, b=b):
            sl = pl.ds(i * L, L)
            val_v[b, sl] = plsc.load_gather(table_v, [idx_v[b, sl]])

        outs[b] = pltpu.async_copy(
            val_v.at[b], out_hbm.at[pl.ds(base + g * CH, CH)], out_sems.at[b]
        )
        if g + NBUF < NCH:
            ins[b] = pltpu.async_copy(
                pos_hbm.at[pl.ds(base + (g + NBUF) * CH, CH)],
                idx_v.at[b],
                in_sems.at[b],
            )
    for b in range(NBUF):
        outs[b].wait()


_mesh = plsc.VectorSubcoreMesh(
    core_axis_name="c", subcore_axis_name="s", num_cores=NC, num_subcores=NS
)

_lookup = pl.kernel(
    _body,
    out_type=jax.ShapeDtypeStruct((N,), jnp.float32),
    mesh=_mesh,
    compiler_params=pltpu.CompilerParams(needs_layout_passes=False),
    scratch_types=[
        pltpu.VMEM((P,), jnp.float32),
        pltpu.VMEM((NBUF, CH), jnp.int32),
        pltpu.VMEM((NBUF, CH), jnp.float32),
        pltpu.SemaphoreType.DMA((NBUF,)),
        pltpu.SemaphoreType.DMA((NBUF,)),
        pltpu.SemaphoreType.DMA,
    ],
)


def kernel(position, W):
    x = position.T.reshape(H // 8, 8, B // 128, 128).swapaxes(1, 2).reshape(N)
    y = _lookup(x, W.reshape(P))
    return y.reshape(H // 8, B // 128, 8, 128).swapaxes(1, 2).reshape(H, B).T


# speedup vs baseline: 905.2320x; 1.0161x over previous
import jax
import jax.numpy as jnp
from jax import lax
from jax.experimental import pallas as pl
from jax.experimental.pallas import tpu as pltpu
from jax.experimental.pallas import tpu_sc as plsc

NC, NS, L = 2, 16, 16
NW = NC * NS
B, H, P = 16384, 200, 200
N = B * H
PER_W = N // NW
NBUF = 4
NCH = 8
CH = PER_W // NCH
UNROLL = 16


def _body(pos_hbm, w_hbm, out_hbm, table_v, idx_v, val_v, in_sems, out_sems, w_sem):
    wid = lax.axis_index("s") * NC + lax.axis_index("c")
    base = wid * PER_W
    w_copy = pltpu.async_copy(w_hbm, table_v, w_sem)

    ins = [
        pltpu.async_copy(
            pos_hbm.at[pl.ds(base + b * CH, CH)], idx_v.at[b], in_sems.at[b]
        )
        for b in range(NBUF)
    ]
    outs = [None] * NBUF
    w_copy.wait()
    for g in range(NCH):
        b = g % NBUF
        ins[b].wait()
        if outs[b] is not None:
            outs[b].wait()

        @plsc.parallel_loop(0, CH // L, unroll=UNROLL)
        def _gather(i, b=b):
            sl = pl.ds(i * L, L)
            val_v[b, sl] = plsc.load_gather(table_v, [idx_v[b, sl]])

        outs[b] = pltpu.async_copy(
            val_v.at[b], out_hbm.at[pl.ds(base + g * CH, CH)], out_sems.at[b]
        )
        if g + NBUF < NCH:
            ins[b] = pltpu.async_copy(
                pos_hbm.at[pl.ds(base + (g + NBUF) * CH, CH)],
                idx_v.at[b],
                in_sems.at[b],
            )
    for b in range(NBUF):
        outs[b].wait()


_mesh = plsc.VectorSubcoreMesh(
    core_axis_name="c", subcore_axis_name="s", num_cores=NC, num_subcores=NS
)

_lookup = pl.kernel(
    _body,
    out_type=jax.ShapeDtypeStruct((N,), jnp.float32),
    mesh=_mesh,
    compiler_params=pltpu.CompilerParams(needs_layout_passes=False),
    scratch_types=[
        pltpu.VMEM((P,), jnp.float32),
        pltpu.VMEM((NBUF, CH), jnp.int32),
        pltpu.VMEM((NBUF, CH), jnp.float32),
        pltpu.SemaphoreType.DMA((NBUF,)),
        pltpu.SemaphoreType.DMA((NBUF,)),
        pltpu.SemaphoreType.DMA,
    ],
)


def kernel(position, W):
    x = position.T.reshape(H // 8, 8, B // 128, 128).swapaxes(1, 2).reshape(N)
    y = _lookup(x, W.reshape(P))
    return y.reshape(H // 8, B // 128, 8, 128).swapaxes(1, 2).reshape(H, B).T
